# baseline (speedup 1.0000x reference)
import jax
import jax.numpy as jnp
from jax import lax
from jax.experimental import pallas as pl
from jax.experimental.pallas import tpu as pltpu
from jax.experimental.pallas import tpu_sc as plsc

N_NODES = 10000
N_EDGES = 320000
D = 128
NW = 32
E_PER_W = N_EDGES // NW
CHUNK = 80
N_CHUNKS = E_PER_W // CHUNK
NBUF = 4
MAIN = 120


def _edge_attn_body(node_hbm, src_hbm, dst_hbm, out_hbm,
                    idx_s, idx_d,
                    rs0, rd0, rs1, rd1, rs2, rd2, rs3, rd3,
                    ss0, sd0, ss1, sd1, ss2, sd2, ss3, sd3,
                    so0, so1, so2, so3):
    wid = lax.axis_index("s") * 2 + lax.axis_index("c")
    w_base = wid * E_PER_W

    rows_s = (rs0, rs1, rs2, rs3)
    rows_d = (rd0, rd1, rd2, rd3)
    sem_s = (ss0, ss1, ss2, ss3)
    sem_d = (sd0, sd1, sd2, sd3)
    sem_o = (so0, so1, so2, so3)

    pltpu.sync_copy(src_hbm.at[pl.ds(w_base, E_PER_W)], idx_s)
    pltpu.sync_copy(dst_hbm.at[pl.ds(w_base, E_PER_W)], idx_d)

    def fire_gather(g, b):
        pltpu.async_copy(node_hbm.at[idx_s.at[pl.ds(g * CHUNK, CHUNK)]],
                         rows_s[b], sem_s[b])
        pltpu.async_copy(node_hbm.at[idx_d.at[pl.ds(g * CHUNK, CHUNK)]],
                         rows_d[b], sem_d[b])

    def drain_gather(b):
        dummy = out_hbm.at[pl.ds(0, CHUNK)]
        pltpu.make_async_copy(dummy, rows_s[b], sem_s[b]).wait()
        pltpu.make_async_copy(dummy, rows_d[b], sem_d[b]).wait()

    def fire_out(g, b):
        pltpu.async_copy(rows_s[b], out_hbm.at[pl.ds(w_base + g * CHUNK, CHUNK)],
                         sem_o[b])

    def drain_out(b):
        dummy = out_hbm.at[pl.ds(0, CHUNK)]
        pltpu.make_async_copy(dummy, rows_s[b], sem_o[b]).wait()

    def mult(b):
        def mul_body(i, _):
            for j in range(D // 16):
                sl = (i, pl.ds(j * 16, 16))
                rows_s[b][sl] = rows_s[b][sl] * rows_d[b][sl]
            return 0

        lax.fori_loop(0, CHUNK, mul_body, 0)

    fire_gather(0, 0)
    fire_gather(1, 1)
    fire_gather(2, 2)

    def steady(g2, _):
        for k in range(NBUF):
            g = g2 * NBUF + k
            drain_gather(k)
            mult(k)
            fire_out(g, k)
            b2 = (k + 3) % NBUF
            if k == 0:
                @pl.when(g2 >= 1)
                def _():
                    drain_out(b2)
            else:
                drain_out(b2)
            fire_gather(g + 3, b2)
        return 0

    lax.fori_loop(0, MAIN // NBUF, steady, 0)

    for g in range(MAIN, N_CHUNKS):
        b = g % NBUF
        drain_gather(b)
        mult(b)
        fire_out(g, b)
        drain_out((g + 3) % NBUF)
        if g + 3 < N_CHUNKS:
            fire_gather(g + 3, (g + 3) % NBUF)
    drain_out((N_CHUNKS - 1) % NBUF)


@jax.jit
def _edge_attn(node_attn, src, dst):
    mesh = plsc.VectorSubcoreMesh(core_axis_name="c", subcore_axis_name="s")
    return pl.kernel(
        _edge_attn_body,
        mesh=mesh,
        out_type=jax.ShapeDtypeStruct((N_EDGES, D), jnp.float32),
        scratch_types=[
            pltpu.VMEM((E_PER_W,), jnp.int32),
            pltpu.VMEM((E_PER_W,), jnp.int32),
        ] + [pltpu.VMEM((CHUNK, D), jnp.float32)] * 8
          + [pltpu.SemaphoreType.DMA] * 12,
    )(node_attn, src, dst)


def kernel(node_attn, edge_index):
    src = edge_index[0].astype(jnp.int32)
    dst = edge_index[1].astype(jnp.int32)
    return _edge_attn(node_attn, src, dst)
